# baseline (speedup 1.0000x reference)
import functools

import jax
import jax.numpy as jnp
import numpy as np
from jax import lax
from jax.experimental import pallas as pl
from jax.experimental.pallas import tpu as pltpu
from jax.experimental.pallas import tpu_sc as plsc

D = 32
NN = 50
NNP = 56
B = 16384

_NC, _NS = 2, 16
NW = _NC * _NS

IDX_W = 128
N_IDX_ROWS = (B * NNP) // IDX_W
ROWS_PER_W = N_IDX_ROWS // NW
FIRE = 8
SUPER = ROWS_PER_W // FIRE
SUPER_ROWS = FIRE * IDX_W
BPW = B // NW
UI_ROWS = BPW // IDX_W


def _sc_gather_body(adj_idx, item_idx, user_idx, entity_tab, item_tab,
                    user_tab, nb_out, item_out, user_out,
                    idx_v, rows_v, gsem):
    wid = lax.axis_index("s") * _NC + lax.axis_index("c")

    pltpu.sync_copy(adj_idx.at[pl.ds(wid * ROWS_PER_W, ROWS_PER_W)], idx_v)

    def super_chunk(sc_i, carry):
        descs = []
        for k in range(FIRE):
            d = pltpu.async_copy(
                entity_tab.at[idx_v.at[sc_i * FIRE + k]],
                rows_v.at[pl.ds(k * IDX_W, IDX_W)],
                gsem)
            descs.append(d)
        for d in descs:
            d.wait()
        pltpu.sync_copy(
            rows_v,
            nb_out.at[pl.ds(wid * ROWS_PER_W * IDX_W + sc_i * SUPER_ROWS,
                            SUPER_ROWS)])
        return carry

    lax.fori_loop(0, SUPER, super_chunk, 0)

    for idx_hbm, tab, out in ((item_idx, item_tab, item_out),
                              (user_idx, user_tab, user_out)):
        pltpu.sync_copy(idx_hbm.at[pl.ds(wid * UI_ROWS, UI_ROWS)],
                        idx_v.at[pl.ds(0, UI_ROWS)])
        descs = []
        for k in range(UI_ROWS):
            descs.append(pltpu.async_copy(
                tab.at[idx_v.at[k]],
                rows_v.at[pl.ds(k * IDX_W, IDX_W)],
                gsem))
        for d in descs:
            d.wait()
        pltpu.sync_copy(rows_v.at[pl.ds(0, BPW)],
                        out.at[pl.ds(wid * BPW, BPW)])


@functools.lru_cache(maxsize=1)
def _sc_gather_fn():
    return pl.kernel(
        _sc_gather_body,
        out_type=(
            jax.ShapeDtypeStruct((B * NNP, D), jnp.float32),
            jax.ShapeDtypeStruct((B, D), jnp.float32),
            jax.ShapeDtypeStruct((B, D), jnp.float32),
        ),
        mesh=plsc.VectorSubcoreMesh(core_axis_name="c", subcore_axis_name="s",
                                    num_cores=_NC, num_subcores=_NS),
        scratch_types=(
            pltpu.VMEM((ROWS_PER_W, IDX_W), jnp.int32),
            pltpu.VMEM((SUPER_ROWS, D), jnp.float32),
            pltpu.SemaphoreType.DMA,
        ),
        compiler_params=pltpu.CompilerParams(use_tc_tiling_on_sc=False),
    )


BBLK = 512
WID = NNP * D

_E_NP = np.zeros((NNP, WID), np.float32)
_F_NP = np.zeros((WID, D), np.float32)
for _n in range(NNP):
    _E_NP[_n, _n * D:(_n + 1) * D] = 1.0
for _j in range(WID):
    _F_NP[_j, _j % D] = 1.0


def _tc_body(nb_ref, item_ref, user_ref, wmask_ref, exp_ref, fold_ref,
             wi_ref, ab_ref, w1_ref, b1_ref, w2_ref, b2_ref,
             cwi_ref, cwr_ref, cb_ref, owu_ref, owf_ref, ob_ref, out_ref):
    f32 = jnp.float32
    dot = functools.partial(jnp.dot, preferred_element_type=f32)
    nb2 = nb_ref[...].reshape(BBLK, WID)
    item = item_ref[...]
    user = user_ref[...]

    c = dot(item, wi_ref[...])
    s = dot(nb2, wmask_ref[...]) + c + ab_ref[0, 0]
    s = jnp.where(s >= 0.0, s, 0.2 * s)
    lane = lax.broadcasted_iota(jnp.int32, (BBLK, NNP), 1)
    s = jnp.where(lane < NN, s, -1e30)
    m = jnp.max(s, axis=1, keepdims=True)
    e = jnp.exp(s - m)
    tot = jnp.sum(e, axis=1, keepdims=True)
    af = dot(e, exp_ref[...])
    na = dot(af * nb2, fold_ref[...]) / tot

    h = jnp.maximum(dot(na, w1_ref[...]) + b1_ref[...], 0.0)
    refined = dot(h, w2_ref[...]) + b2_ref[...]
    comb = jnp.maximum(
        dot(item, cwi_ref[...]) + dot(refined, cwr_ref[...]) + cb_ref[...],
        0.0)
    score = (dot(user, owu_ref[...]) + dot(comb, owf_ref[...])
             + ob_ref[0, 0])
    out_ref[...] = score


def _tc_compute(nb3, item_emb, user_emb, wmask, wi, ab, w1, b1, w2, b2,
                cwi, cwr, cb, owu, owf, ob):
    n_blocks = B // BBLK
    small = lambda shp: pl.BlockSpec(shp, lambda i: (0, 0))
    return pl.pallas_call(
        _tc_body,
        grid=(n_blocks,),
        in_specs=[
            pl.BlockSpec((BBLK, NNP, D), lambda i: (i, 0, 0)),
            pl.BlockSpec((BBLK, D), lambda i: (i, 0)),
            pl.BlockSpec((BBLK, D), lambda i: (i, 0)),
            small((WID, NNP)), small((NNP, WID)), small((WID, D)),
            small((D, 1)), small((1, 1)),
            small((D, D)), small((1, D)), small((D, D)), small((1, D)),
            small((D, D)), small((D, D)), small((1, D)),
            small((D, 1)), small((D, 1)), small((1, 1)),
        ],
        out_specs=pl.BlockSpec((BBLK, 1), lambda i: (i, 0)),
        out_shape=jax.ShapeDtypeStruct((B, 1), jnp.float32),
    )(nb3, item_emb, user_emb, wmask, jnp.asarray(_E_NP), jnp.asarray(_F_NP),
      wi, ab, w1, b1, w2, b2, cwi, cwr, cb, owu, owf, ob)


def kernel(user_indices, item_indices, kg_adj_batch, user_table, item_table,
           entity_table, attn_W, attn_b, kg_W1, kg_b1, kg_W2, kg_b2,
           comb_W, comb_b, out_W, out_b):
    adj = jnp.maximum(kg_adj_batch, 0).astype(jnp.int32)
    adj_p = jnp.pad(adj, ((0, 0), (0, NNP - NN)))
    adj2 = adj_p.reshape(N_IDX_ROWS, IDX_W)
    ii = item_indices.astype(jnp.int32).reshape(B // IDX_W, IDX_W)
    ui = user_indices.astype(jnp.int32).reshape(B // IDX_W, IDX_W)

    nb_flat, item_emb, user_emb = _sc_gather_fn()(
        adj2, ii, ui, entity_table, item_table, user_table)
    nb3 = nb_flat.reshape(B, NNP, D)

    wn_tiled = jnp.tile(attn_W[D:, 0], NNP)
    wmask = jnp.asarray(_E_NP).T * wn_tiled[:, None]
    wi = attn_W[:D]
    ab = attn_b.reshape(1, 1)
    b1 = kg_b1.reshape(1, D)
    b2 = kg_b2.reshape(1, D)
    cwi = comb_W[:D]
    cwr = comb_W[D:]
    cb = comb_b.reshape(1, D)
    owu = out_W[:D]
    owf = out_W[D:]
    ob = out_b.reshape(1, 1)

    score = _tc_compute(nb3, item_emb, user_emb, wmask, wi, ab, kg_W1, b1,
                        kg_W2, b2, cwi, cwr, cb, owu, owf, ob)
    return score[:, 0]


# speedup vs baseline: 1.9313x; 1.9313x over previous
import functools

import jax
import jax.numpy as jnp
import numpy as np
from jax import lax
from jax.experimental import pallas as pl
from jax.experimental.pallas import tpu as pltpu
from jax.experimental.pallas import tpu_sc as plsc

D = 32
NN = 50
NNP = 56
B = 16384

_NC, _NS = 2, 16
NW = _NC * _NS

IDX_W = 128
N_IDX_ROWS = (B * NNP) // IDX_W
ROWS_PER_W = N_IDX_ROWS // NW
FIRE = 8
SUPER = ROWS_PER_W // FIRE
SUPER_ROWS = FIRE * IDX_W
BPW = B // NW
UI_ROWS = BPW // IDX_W


def _sc_gather_body(adj_idx, item_idx, user_idx, entity_tab, item_tab,
                    user_tab, nb_out, item_out, user_out,
                    idx_v, rows_v, gsem):
    wid = lax.axis_index("s") * _NC + lax.axis_index("c")

    pltpu.sync_copy(adj_idx.at[pl.ds(wid * ROWS_PER_W, ROWS_PER_W)], idx_v)

    def super_chunk(sc_i, carry):
        descs = []
        for k in range(FIRE):
            d = pltpu.async_copy(
                entity_tab.at[idx_v.at[sc_i * FIRE + k]],
                rows_v.at[pl.ds(k * IDX_W, IDX_W)],
                gsem)
            descs.append(d)
        for d in descs:
            d.wait()
        pltpu.sync_copy(
            rows_v,
            nb_out.at[pl.ds(wid * ROWS_PER_W * IDX_W + sc_i * SUPER_ROWS,
                            SUPER_ROWS)])
        return carry

    lax.fori_loop(0, SUPER, super_chunk, 0)

    for idx_hbm, tab, out in ((item_idx, item_tab, item_out),
                              (user_idx, user_tab, user_out)):
        pltpu.sync_copy(idx_hbm.at[pl.ds(wid * UI_ROWS, UI_ROWS)],
                        idx_v.at[pl.ds(0, UI_ROWS)])
        descs = []
        for k in range(UI_ROWS):
            descs.append(pltpu.async_copy(
                tab.at[idx_v.at[k]],
                rows_v.at[pl.ds(k * IDX_W, IDX_W)],
                gsem))
        for d in descs:
            d.wait()
        pltpu.sync_copy(rows_v.at[pl.ds(0, BPW)],
                        out.at[pl.ds(wid * BPW, BPW)])


@functools.lru_cache(maxsize=1)
def _sc_gather_fn():
    return pl.kernel(
        _sc_gather_body,
        out_type=(
            jax.ShapeDtypeStruct((B * NNP, D), jnp.float32),
            jax.ShapeDtypeStruct((B, D), jnp.float32),
            jax.ShapeDtypeStruct((B, D), jnp.float32),
        ),
        mesh=plsc.VectorSubcoreMesh(core_axis_name="c", subcore_axis_name="s",
                                    num_cores=_NC, num_subcores=_NS),
        scratch_types=(
            pltpu.VMEM((ROWS_PER_W, IDX_W), jnp.int32),
            pltpu.VMEM((SUPER_ROWS, D), jnp.float32),
            pltpu.SemaphoreType.DMA,
        ),
        compiler_params=pltpu.CompilerParams(use_tc_tiling_on_sc=False),
    )


BBLK = 512
WID = NNP * D

_E_NP = np.zeros((NNP, WID), np.float32)
_F_NP = np.zeros((WID, D), np.float32)
for _n in range(NNP):
    _E_NP[_n, _n * D:(_n + 1) * D] = 1.0
for _j in range(WID):
    _F_NP[_j, _j % D] = 1.0


def _tc_body(nb_ref, item_ref, user_ref, wmask_ref, exp_ref, fold_ref,
             wi_ref, ab_ref, w1_ref, b1_ref, w2_ref, b2_ref,
             cwi_ref, cwr_ref, cb_ref, owu_ref, owf_ref, ob_ref, out_ref):
    f32 = jnp.float32
    dot = functools.partial(jnp.dot, preferred_element_type=f32)
    nb2 = nb_ref[...].reshape(BBLK, WID)
    item = item_ref[...]
    user = user_ref[...]

    c = dot(item, wi_ref[...])
    s = dot(nb2, wmask_ref[...]) + c + ab_ref[0, 0]
    s = jnp.where(s >= 0.0, s, 0.2 * s)
    lane = lax.broadcasted_iota(jnp.int32, (BBLK, NNP), 1)
    s = jnp.where(lane < NN, s, -1e30)
    m = jnp.max(s, axis=1, keepdims=True)
    e = jnp.exp(s - m)
    tot = jnp.sum(e, axis=1, keepdims=True)
    af = dot(e, exp_ref[...])
    na = dot(af * nb2, fold_ref[...]) / tot

    h = jnp.maximum(dot(na, w1_ref[...]) + b1_ref[...], 0.0)
    refined = dot(h, w2_ref[...]) + b2_ref[...]
    comb = jnp.maximum(
        dot(item, cwi_ref[...]) + dot(refined, cwr_ref[...]) + cb_ref[...],
        0.0)
    score = (dot(user, owu_ref[...]) + dot(comb, owf_ref[...])
             + ob_ref[0, 0])
    out_ref[...] = score


def _tc_compute(nb3, item_emb, user_emb, wmask, wi, ab, w1, b1, w2, b2,
                cwi, cwr, cb, owu, owf, ob):
    n_blocks = B // BBLK
    small = lambda shp: pl.BlockSpec(shp, lambda i: (0, 0))
    return pl.pallas_call(
        _tc_body,
        grid=(n_blocks,),
        in_specs=[
            pl.BlockSpec((BBLK, NNP, D), lambda i: (i, 0, 0)),
            pl.BlockSpec((BBLK, D), lambda i: (i, 0)),
            pl.BlockSpec((BBLK, D), lambda i: (i, 0)),
            small((WID, NNP)), small((NNP, WID)), small((WID, D)),
            small((D, 1)), small((1, 1)),
            small((D, D)), small((1, D)), small((D, D)), small((1, D)),
            small((D, D)), small((D, D)), small((1, D)),
            small((D, 1)), small((D, 1)), small((1, 1)),
        ],
        out_specs=pl.BlockSpec((BBLK, 1), lambda i: (i, 0)),
        out_shape=jax.ShapeDtypeStruct((B, 1), jnp.float32),
    )(nb3, item_emb, user_emb, wmask, jnp.asarray(_E_NP), jnp.asarray(_F_NP),
      wi, ab, w1, b1, w2, b2, cwi, cwr, cb, owu, owf, ob)


def kernel(user_indices, item_indices, kg_adj_batch, user_table, item_table,
           entity_table, attn_W, attn_b, kg_W1, kg_b1, kg_W2, kg_b2,
           comb_W, comb_b, out_W, out_b):
    adj = jnp.maximum(kg_adj_batch, 0).astype(jnp.int32)
    n_ent = entity_table.shape[0]
    filler = (lax.broadcasted_iota(jnp.int32, (B, NNP - NN), 0) * (NNP - NN)
              + lax.broadcasted_iota(jnp.int32, (B, NNP - NN), 1)) % n_ent
    adj_p = jnp.concatenate([adj, filler], axis=1)
    adj2 = adj_p.reshape(N_IDX_ROWS, IDX_W)
    ii = item_indices.astype(jnp.int32).reshape(B // IDX_W, IDX_W)
    ui = user_indices.astype(jnp.int32).reshape(B // IDX_W, IDX_W)

    nb_flat, item_emb, user_emb = _sc_gather_fn()(
        adj2, ii, ui, entity_table, item_table, user_table)
    nb3 = nb_flat.reshape(B, NNP, D)

    wn_tiled = jnp.tile(attn_W[D:, 0], NNP)
    wmask = jnp.asarray(_E_NP).T * wn_tiled[:, None]
    wi = attn_W[:D]
    ab = attn_b.reshape(1, 1)
    b1 = kg_b1.reshape(1, D)
    b2 = kg_b2.reshape(1, D)
    cwi = comb_W[:D]
    cwr = comb_W[D:]
    cb = comb_b.reshape(1, D)
    owu = out_W[:D]
    owf = out_W[D:]
    ob = out_b.reshape(1, 1)

    score = _tc_compute(nb3, item_emb, user_emb, wmask, wi, ab, kg_W1, b1,
                        kg_W2, b2, cwi, cwr, cb, owu, owf, ob)
    return score[:, 0]


# speedup vs baseline: 1.9316x; 1.0002x over previous
import functools

import jax
import jax.numpy as jnp
import numpy as np
from jax import lax
from jax.experimental import pallas as pl
from jax.experimental.pallas import tpu as pltpu
from jax.experimental.pallas import tpu_sc as plsc

D = 32
NN = 50
NNP = 56
B = 16384

_NC, _NS = 2, 16
NW = _NC * _NS

IDX_W = 128
N_IDX_ROWS = (B * NNP) // IDX_W
ROWS_PER_W = N_IDX_ROWS // NW
FIRE = 8
SUPER = ROWS_PER_W // FIRE
SUPER_ROWS = FIRE * IDX_W
BPW = B // NW
UI_ROWS = BPW // IDX_W


def _sc_gather_body(adj_idx, item_idx, user_idx, entity_tab, item_tab,
                    user_tab, nb_out, item_out, user_out,
                    idx_v, rows_v, gsem):
    wid = lax.axis_index("s") * _NC + lax.axis_index("c")

    pltpu.sync_copy(adj_idx.at[pl.ds(wid * ROWS_PER_W, ROWS_PER_W)], idx_v)

    def super_chunk(sc_i, carry):
        descs = []
        for k in range(FIRE):
            d = pltpu.async_copy(
                entity_tab.at[idx_v.at[sc_i * FIRE + k]],
                rows_v.at[pl.ds(k * IDX_W, IDX_W)],
                gsem)
            descs.append(d)
        for d in descs:
            d.wait()
        pltpu.sync_copy(
            rows_v,
            nb_out.at[pl.ds(wid * ROWS_PER_W * IDX_W + sc_i * SUPER_ROWS,
                            SUPER_ROWS)])
        return carry

    lax.fori_loop(0, SUPER, super_chunk, 0)

    for idx_hbm, tab, out in ((item_idx, item_tab, item_out),
                              (user_idx, user_tab, user_out)):
        pltpu.sync_copy(idx_hbm.at[pl.ds(wid * UI_ROWS, UI_ROWS)],
                        idx_v.at[pl.ds(0, UI_ROWS)])
        descs = []
        for k in range(UI_ROWS):
            descs.append(pltpu.async_copy(
                tab.at[idx_v.at[k]],
                rows_v.at[pl.ds(k * IDX_W, IDX_W)],
                gsem))
        for d in descs:
            d.wait()
        pltpu.sync_copy(rows_v.at[pl.ds(0, BPW)],
                        out.at[pl.ds(wid * BPW, BPW)])


@functools.lru_cache(maxsize=1)
def _sc_gather_fn():
    return pl.kernel(
        _sc_gather_body,
        out_type=(
            jax.ShapeDtypeStruct((B * NNP, D), jnp.float32),
            jax.ShapeDtypeStruct((B, D), jnp.float32),
            jax.ShapeDtypeStruct((B, D), jnp.float32),
        ),
        mesh=plsc.VectorSubcoreMesh(core_axis_name="c", subcore_axis_name="s",
                                    num_cores=_NC, num_subcores=_NS),
        scratch_types=(
            pltpu.VMEM((ROWS_PER_W, IDX_W), jnp.int32),
            pltpu.VMEM((SUPER_ROWS, D), jnp.float32),
            pltpu.SemaphoreType.DMA,
        ),
        compiler_params=pltpu.CompilerParams(use_tc_tiling_on_sc=False),
    )


BBLK = 512
WID = NNP * D

_E_NP = np.zeros((NNP, WID), np.float32)
_F_NP = np.zeros((WID, D), np.float32)
for _n in range(NNP):
    _E_NP[_n, _n * D:(_n + 1) * D] = 1.0
for _j in range(WID):
    _F_NP[_j, _j % D] = 1.0


def _tc_body(nb_ref, item_ref, user_ref, wmask_ref, exp_ref, fold_ref,
             wi_ref, ab_ref, w1_ref, b1_ref, w2_ref, b2_ref,
             cwi_ref, cwr_ref, cb_ref, owu_ref, owf_ref, ob_ref, out_ref):
    f32 = jnp.float32
    dot = functools.partial(jnp.dot, preferred_element_type=f32)
    nb2 = nb_ref.reshape(BBLK, NNP, D)[...].reshape(BBLK, WID)
    item = item_ref[...]
    user = user_ref[...]

    c = dot(item, wi_ref[...])
    s = dot(nb2, wmask_ref[...]) + c + ab_ref[0, 0]
    s = jnp.where(s >= 0.0, s, 0.2 * s)
    lane = lax.broadcasted_iota(jnp.int32, (BBLK, NNP), 1)
    s = jnp.where(lane < NN, s, -1e30)
    m = jnp.max(s, axis=1, keepdims=True)
    e = jnp.exp(s - m)
    tot = jnp.sum(e, axis=1, keepdims=True)
    af = dot(e, exp_ref[...])
    na = dot(af * nb2, fold_ref[...]) / tot

    h = jnp.maximum(dot(na, w1_ref[...]) + b1_ref[...], 0.0)
    refined = dot(h, w2_ref[...]) + b2_ref[...]
    comb = jnp.maximum(
        dot(item, cwi_ref[...]) + dot(refined, cwr_ref[...]) + cb_ref[...],
        0.0)
    score = (dot(user, owu_ref[...]) + dot(comb, owf_ref[...])
             + ob_ref[0, 0])
    out_ref[...] = score


def _tc_compute(nb_flat, item_emb, user_emb, wmask, wi, ab, w1, b1, w2, b2,
                cwi, cwr, cb, owu, owf, ob):
    n_blocks = B // BBLK
    small = lambda shp: pl.BlockSpec(shp, lambda i: (0, 0))
    return pl.pallas_call(
        _tc_body,
        grid=(n_blocks,),
        in_specs=[
            pl.BlockSpec((BBLK * NNP, D), lambda i: (i, 0)),
            pl.BlockSpec((BBLK, D), lambda i: (i, 0)),
            pl.BlockSpec((BBLK, D), lambda i: (i, 0)),
            small((WID, NNP)), small((NNP, WID)), small((WID, D)),
            small((D, 1)), small((1, 1)),
            small((D, D)), small((1, D)), small((D, D)), small((1, D)),
            small((D, D)), small((D, D)), small((1, D)),
            small((D, 1)), small((D, 1)), small((1, 1)),
        ],
        out_specs=pl.BlockSpec((BBLK, 1), lambda i: (i, 0)),
        out_shape=jax.ShapeDtypeStruct((B, 1), jnp.float32),
    )(nb_flat, item_emb, user_emb, wmask, jnp.asarray(_E_NP), jnp.asarray(_F_NP),
      wi, ab, w1, b1, w2, b2, cwi, cwr, cb, owu, owf, ob)


def kernel(user_indices, item_indices, kg_adj_batch, user_table, item_table,
           entity_table, attn_W, attn_b, kg_W1, kg_b1, kg_W2, kg_b2,
           comb_W, comb_b, out_W, out_b):
    adj = jnp.maximum(kg_adj_batch, 0).astype(jnp.int32)
    n_ent = entity_table.shape[0]
    filler = (lax.broadcasted_iota(jnp.int32, (B, NNP - NN), 0) * (NNP - NN)
              + lax.broadcasted_iota(jnp.int32, (B, NNP - NN), 1)) % n_ent
    adj_p = jnp.concatenate([adj, filler], axis=1)
    adj2 = adj_p.reshape(N_IDX_ROWS, IDX_W)
    ii = item_indices.astype(jnp.int32).reshape(B // IDX_W, IDX_W)
    ui = user_indices.astype(jnp.int32).reshape(B // IDX_W, IDX_W)

    nb_flat, item_emb, user_emb = _sc_gather_fn()(
        adj2, ii, ui, entity_table, item_table, user_table)

    wn_tiled = jnp.tile(attn_W[D:, 0], NNP)
    wmask = jnp.asarray(_E_NP).T * wn_tiled[:, None]
    wi = attn_W[:D]
    ab = attn_b.reshape(1, 1)
    b1 = kg_b1.reshape(1, D)
    b2 = kg_b2.reshape(1, D)
    cwi = comb_W[:D]
    cwr = comb_W[D:]
    cb = comb_b.reshape(1, D)
    owu = out_W[:D]
    owf = out_W[D:]
    ob = out_b.reshape(1, 1)

    score = _tc_compute(nb_flat, item_emb, user_emb, wmask, wi, ab, kg_W1, b1,
                        kg_W2, b2, cwi, cwr, cb, owu, owf, ob)
    return score[:, 0]


# speedup vs baseline: 2.5082x; 1.2985x over previous
import functools

import jax
import jax.numpy as jnp
import numpy as np
from jax import lax
from jax.experimental import pallas as pl
from jax.experimental.pallas import tpu as pltpu
from jax.experimental.pallas import tpu_sc as plsc

D = 32
NN = 50
NNP = 56
B = 16384

_NC, _NS = 2, 16
NW = _NC * _NS

IDX_W = 128
N_IDX_ROWS = (B * NNP) // IDX_W
ROWS_PER_W = N_IDX_ROWS // NW
FIRE = 8
SUPER = ROWS_PER_W // FIRE
SUPER_ROWS = FIRE * IDX_W
BPW = B // NW
UI_ROWS = BPW // IDX_W


def _sc_gather_body(adj_idx, item_idx, user_idx, entity_tab, item_tab,
                    user_tab, nb_out, item_out, user_out,
                    idx_v, rows_v, gsem):
    wid = lax.axis_index("s") * _NC + lax.axis_index("c")

    pltpu.sync_copy(adj_idx.at[pl.ds(wid * ROWS_PER_W, ROWS_PER_W)], idx_v)

    def super_chunk(sc_i, carry):
        descs = []
        for k in range(FIRE):
            d = pltpu.async_copy(
                entity_tab.at[idx_v.at[sc_i * FIRE + k]],
                rows_v.at[pl.ds(k * IDX_W, IDX_W)],
                gsem)
            descs.append(d)
        for d in descs:
            d.wait()
        pltpu.sync_copy(
            rows_v,
            nb_out.at[pl.ds(wid * ROWS_PER_W * IDX_W + sc_i * SUPER_ROWS,
                            SUPER_ROWS)])
        return carry

    lax.fori_loop(0, SUPER, super_chunk, 0)

    for idx_hbm, tab, out in ((item_idx, item_tab, item_out),
                              (user_idx, user_tab, user_out)):
        pltpu.sync_copy(idx_hbm.at[pl.ds(wid * UI_ROWS, UI_ROWS)],
                        idx_v.at[pl.ds(0, UI_ROWS)])
        descs = []
        for k in range(UI_ROWS):
            descs.append(pltpu.async_copy(
                tab.at[idx_v.at[k]],
                rows_v.at[pl.ds(k * IDX_W, IDX_W)],
                gsem))
        for d in descs:
            d.wait()
        pltpu.sync_copy(rows_v.at[pl.ds(0, BPW)],
                        out.at[pl.ds(wid * BPW, BPW)])


@functools.lru_cache(maxsize=1)
def _sc_gather_fn():
    return pl.kernel(
        _sc_gather_body,
        out_type=(
            jax.ShapeDtypeStruct((B * NNP, D), jnp.float32),
            jax.ShapeDtypeStruct((B, D), jnp.float32),
            jax.ShapeDtypeStruct((B, D), jnp.float32),
        ),
        mesh=plsc.VectorSubcoreMesh(core_axis_name="c", subcore_axis_name="s",
                                    num_cores=_NC, num_subcores=_NS),
        scratch_types=(
            pltpu.VMEM((ROWS_PER_W, IDX_W), jnp.int32),
            pltpu.VMEM((SUPER_ROWS, D), jnp.float32),
            pltpu.SemaphoreType.DMA,
        ),
        compiler_params=pltpu.CompilerParams(use_tc_tiling_on_sc=False),
    )


BBLK = 512
WID = NNP * D

_E_NP = np.zeros((NNP, WID), np.float32)
_F_NP = np.zeros((WID, D), np.float32)
for _n in range(NNP):
    _E_NP[_n, _n * D:(_n + 1) * D] = 1.0
for _j in range(WID):
    _F_NP[_j, _j % D] = 1.0


def _tc_body(nb_ref, item_ref, user_ref, wmask_ref, exp_ref, fold_ref,
             wi_ref, ab_ref, w1_ref, b1_ref, w2_ref, b2_ref,
             cwi_ref, cwr_ref, cb_ref, owu_ref, owf_ref, ob_ref, out_ref):
    f32 = jnp.float32
    dot = functools.partial(jnp.dot, preferred_element_type=f32)
    nb2 = nb_ref[...]
    item = item_ref[...]
    user = user_ref[...]

    c = dot(item, wi_ref[...])
    s = dot(nb2, wmask_ref[...]) + c + ab_ref[0, 0]
    s = jnp.where(s >= 0.0, s, 0.2 * s)
    lane = lax.broadcasted_iota(jnp.int32, (BBLK, NNP), 1)
    s = jnp.where(lane < NN, s, -1e30)
    m = jnp.max(s, axis=1, keepdims=True)
    e = jnp.exp(s - m)
    tot = jnp.sum(e, axis=1, keepdims=True)
    af = dot(e, exp_ref[...])
    na = dot(af * nb2, fold_ref[...]) / tot

    h = jnp.maximum(dot(na, w1_ref[...]) + b1_ref[...], 0.0)
    refined = dot(h, w2_ref[...]) + b2_ref[...]
    comb = jnp.maximum(
        dot(item, cwi_ref[...]) + dot(refined, cwr_ref[...]) + cb_ref[...],
        0.0)
    score = (dot(user, owu_ref[...]) + dot(comb, owf_ref[...])
             + ob_ref[0, 0])
    out_ref[...] = score


def _tc_compute(nb2, item_emb, user_emb, wmask, wi, ab, w1, b1, w2, b2,
                cwi, cwr, cb, owu, owf, ob):
    n_blocks = B // BBLK
    small = lambda shp: pl.BlockSpec(shp, lambda i: (0, 0))
    return pl.pallas_call(
        _tc_body,
        grid=(n_blocks,),
        in_specs=[
            pl.BlockSpec((BBLK, WID), lambda i: (i, 0)),
            pl.BlockSpec((BBLK, D), lambda i: (i, 0)),
            pl.BlockSpec((BBLK, D), lambda i: (i, 0)),
            small((WID, NNP)), small((NNP, WID)), small((WID, D)),
            small((D, 1)), small((1, 1)),
            small((D, D)), small((1, D)), small((D, D)), small((1, D)),
            small((D, D)), small((D, D)), small((1, D)),
            small((D, 1)), small((D, 1)), small((1, 1)),
        ],
        out_specs=pl.BlockSpec((BBLK, 1), lambda i: (i, 0)),
        out_shape=jax.ShapeDtypeStruct((B, 1), jnp.float32),
    )(nb2, item_emb, user_emb, wmask, jnp.asarray(_E_NP), jnp.asarray(_F_NP),
      wi, ab, w1, b1, w2, b2, cwi, cwr, cb, owu, owf, ob)


def kernel(user_indices, item_indices, kg_adj_batch, user_table, item_table,
           entity_table, attn_W, attn_b, kg_W1, kg_b1, kg_W2, kg_b2,
           comb_W, comb_b, out_W, out_b):
    adj = jnp.maximum(kg_adj_batch, 0).astype(jnp.int32)
    n_ent = entity_table.shape[0]
    filler = (lax.broadcasted_iota(jnp.int32, (B, NNP - NN), 0) * (NNP - NN)
              + lax.broadcasted_iota(jnp.int32, (B, NNP - NN), 1)) % n_ent
    adj_p = jnp.concatenate([adj, filler], axis=1)
    adj2 = adj_p.reshape(N_IDX_ROWS, IDX_W)
    ii = item_indices.astype(jnp.int32).reshape(B // IDX_W, IDX_W)
    ui = user_indices.astype(jnp.int32).reshape(B // IDX_W, IDX_W)

    nb_flat, item_emb, user_emb = _sc_gather_fn()(
        adj2, ii, ui, entity_table, item_table, user_table)

    wn_tiled = jnp.tile(attn_W[D:, 0], NNP)
    wmask = jnp.asarray(_E_NP).T * wn_tiled[:, None]
    wi = attn_W[:D]
    ab = attn_b.reshape(1, 1)
    b1 = kg_b1.reshape(1, D)
    b2 = kg_b2.reshape(1, D)
    cwi = comb_W[:D]
    cwr = comb_W[D:]
    cb = comb_b.reshape(1, D)
    owu = out_W[:D]
    owf = out_W[D:]
    ob = out_b.reshape(1, 1)

    score = _tc_compute(nb_flat.reshape(B, WID), item_emb, user_emb, wmask, wi, ab, kg_W1, b1,
                        kg_W2, b2, cwi, cwr, cb, owu, owf, ob)
    return score[:, 0]


# speedup vs baseline: 2.5561x; 1.0191x over previous
import functools

import jax
import jax.numpy as jnp
import numpy as np
from jax import lax
from jax.experimental import pallas as pl
from jax.experimental.pallas import tpu as pltpu
from jax.experimental.pallas import tpu_sc as plsc

D = 32
NN = 50
NNP = 56
B = 16384

_NC, _NS = 2, 16
NW = _NC * _NS

NCHUNK = 2
BC = B // NCHUNK
IDX_W = 128
N_IDX_ROWS = (BC * NNP) // IDX_W
ROWS_PER_W = N_IDX_ROWS // NW
FIRE = 8
SUPER = ROWS_PER_W // FIRE
SUPER_ROWS = FIRE * IDX_W
BPW = BC // NW
UI_ROWS = BPW // IDX_W


def _sc_gather_body(adj_idx, item_idx, user_idx, entity_tab, item_tab,
                    user_tab, nb_out, item_out, user_out,
                    idx_v, rows_v, gsem):
    wid = lax.axis_index("s") * _NC + lax.axis_index("c")

    pltpu.sync_copy(adj_idx.at[pl.ds(wid * ROWS_PER_W, ROWS_PER_W)], idx_v)

    def super_chunk(sc_i, carry):
        descs = []
        for k in range(FIRE):
            d = pltpu.async_copy(
                entity_tab.at[idx_v.at[sc_i * FIRE + k]],
                rows_v.at[pl.ds(k * IDX_W, IDX_W)],
                gsem)
            descs.append(d)
        for d in descs:
            d.wait()
        pltpu.sync_copy(
            rows_v,
            nb_out.at[pl.ds(wid * ROWS_PER_W * IDX_W + sc_i * SUPER_ROWS,
                            SUPER_ROWS)])
        return carry

    lax.fori_loop(0, SUPER, super_chunk, 0)

    for idx_hbm, tab, out in ((item_idx, item_tab, item_out),
                              (user_idx, user_tab, user_out)):
        pltpu.sync_copy(idx_hbm.at[pl.ds(wid * UI_ROWS, UI_ROWS)],
                        idx_v.at[pl.ds(0, UI_ROWS)])
        descs = []
        for k in range(UI_ROWS):
            descs.append(pltpu.async_copy(
                tab.at[idx_v.at[k]],
                rows_v.at[pl.ds(k * IDX_W, IDX_W)],
                gsem))
        for d in descs:
            d.wait()
        pltpu.sync_copy(rows_v.at[pl.ds(0, BPW)],
                        out.at[pl.ds(wid * BPW, BPW)])


@functools.lru_cache(maxsize=1)
def _sc_gather_fn():
    return pl.kernel(
        _sc_gather_body,
        out_type=(
            jax.ShapeDtypeStruct((BC * NNP, D), jnp.float32),
            jax.ShapeDtypeStruct((BC, D), jnp.float32),
            jax.ShapeDtypeStruct((BC, D), jnp.float32),
        ),
        mesh=plsc.VectorSubcoreMesh(core_axis_name="c", subcore_axis_name="s",
                                    num_cores=_NC, num_subcores=_NS),
        scratch_types=(
            pltpu.VMEM((ROWS_PER_W, IDX_W), jnp.int32),
            pltpu.VMEM((SUPER_ROWS, D), jnp.float32),
            pltpu.SemaphoreType.DMA,
        ),
        compiler_params=pltpu.CompilerParams(use_tc_tiling_on_sc=False),
    )


BBLK = 512
WID = NNP * D

_E_NP = np.zeros((NNP, WID), np.float32)
_F_NP = np.zeros((WID, D), np.float32)
for _n in range(NNP):
    _E_NP[_n, _n * D:(_n + 1) * D] = 1.0
for _j in range(WID):
    _F_NP[_j, _j % D] = 1.0


def _tc_body(nb_ref, item_ref, user_ref, wmask_ref, exp_ref, fold_ref,
             wi_ref, ab_ref, w1_ref, b1_ref, w2_ref, b2_ref,
             cwi_ref, cwr_ref, cb_ref, owu_ref, owf_ref, ob_ref, out_ref):
    f32 = jnp.float32
    dot = functools.partial(jnp.dot, preferred_element_type=f32)
    nb2 = nb_ref[...]
    item = item_ref[...]
    user = user_ref[...]

    c = dot(item, wi_ref[...])
    s = dot(nb2, wmask_ref[...]) + c + ab_ref[0, 0]
    s = jnp.where(s >= 0.0, s, 0.2 * s)
    lane = lax.broadcasted_iota(jnp.int32, (BBLK, NNP), 1)
    s = jnp.where(lane < NN, s, -1e30)
    m = jnp.max(s, axis=1, keepdims=True)
    e = jnp.exp(s - m)
    tot = jnp.sum(e, axis=1, keepdims=True)
    af = dot(e, exp_ref[...])
    na = dot(af * nb2, fold_ref[...]) / tot

    h = jnp.maximum(dot(na, w1_ref[...]) + b1_ref[...], 0.0)
    refined = dot(h, w2_ref[...]) + b2_ref[...]
    comb = jnp.maximum(
        dot(item, cwi_ref[...]) + dot(refined, cwr_ref[...]) + cb_ref[...],
        0.0)
    score = (dot(user, owu_ref[...]) + dot(comb, owf_ref[...])
             + ob_ref[0, 0])
    out_ref[...] = score


def _tc_compute(nb2, item_emb, user_emb, wmask, wi, ab, w1, b1, w2, b2,
                cwi, cwr, cb, owu, owf, ob):
    n_blocks = BC // BBLK
    small = lambda shp: pl.BlockSpec(shp, lambda i: (0, 0))
    return pl.pallas_call(
        _tc_body,
        grid=(n_blocks,),
        in_specs=[
            pl.BlockSpec((BBLK, WID), lambda i: (i, 0)),
            pl.BlockSpec((BBLK, D), lambda i: (i, 0)),
            pl.BlockSpec((BBLK, D), lambda i: (i, 0)),
            small((WID, NNP)), small((NNP, WID)), small((WID, D)),
            small((D, 1)), small((1, 1)),
            small((D, D)), small((1, D)), small((D, D)), small((1, D)),
            small((D, D)), small((D, D)), small((1, D)),
            small((D, 1)), small((D, 1)), small((1, 1)),
        ],
        out_specs=pl.BlockSpec((BBLK, 1), lambda i: (i, 0)),
        out_shape=jax.ShapeDtypeStruct((BC, 1), jnp.float32),
    )(nb2, item_emb, user_emb, wmask, jnp.asarray(_E_NP), jnp.asarray(_F_NP),
      wi, ab, w1, b1, w2, b2, cwi, cwr, cb, owu, owf, ob)


def kernel(user_indices, item_indices, kg_adj_batch, user_table, item_table,
           entity_table, attn_W, attn_b, kg_W1, kg_b1, kg_W2, kg_b2,
           comb_W, comb_b, out_W, out_b):
    adj = jnp.maximum(kg_adj_batch, 0).astype(jnp.int32)
    n_ent = entity_table.shape[0]
    filler = (lax.broadcasted_iota(jnp.int32, (B, NNP - NN), 0) * (NNP - NN)
              + lax.broadcasted_iota(jnp.int32, (B, NNP - NN), 1)) % n_ent
    adj_p = jnp.concatenate([adj, filler], axis=1)

    wn_tiled = jnp.tile(attn_W[D:, 0], NNP)
    wmask = jnp.asarray(_E_NP).T * wn_tiled[:, None]
    wi = attn_W[:D]
    ab = attn_b.reshape(1, 1)
    b1 = kg_b1.reshape(1, D)
    b2 = kg_b2.reshape(1, D)
    cwi = comb_W[:D]
    cwr = comb_W[D:]
    cb = comb_b.reshape(1, D)
    owu = out_W[:D]
    owf = out_W[D:]
    ob = out_b.reshape(1, 1)

    gather = _sc_gather_fn()
    scores = []
    for k in range(NCHUNK):
        sl = slice(k * BC, (k + 1) * BC)
        adj2 = adj_p[sl].reshape(N_IDX_ROWS, IDX_W)
        ii = item_indices[sl].astype(jnp.int32).reshape(BC // IDX_W, IDX_W)
        ui = user_indices[sl].astype(jnp.int32).reshape(BC // IDX_W, IDX_W)
        nb_flat, item_emb, user_emb = gather(
            adj2, ii, ui, entity_table, item_table, user_table)
        scores.append(_tc_compute(
            nb_flat.reshape(BC, WID), item_emb, user_emb, wmask, wi, ab,
            kg_W1, b1, kg_W2, b2, cwi, cwr, cb, owu, owf, ob))
    return jnp.concatenate(scores, axis=0)[:, 0]
